# baseline (speedup 1.0000x reference)
import functools

import jax
import jax.numpy as jnp
from jax.experimental import pallas as pl
from jax.experimental.pallas import tpu as pltpu
from jax.experimental.pallas import tpu_sc as plsc

NUM_EXPERTS = 8
D_MODEL = 1024
D_FF = 2048
D_OUT = 1024
N_TOK = 2048

BLK = 512
NBLK = 11
NPAD = NBLK * BLK


def _router_body(x_ref, wr_ref, br_ref,
                 logits_ref, probs_ref, routes_ref, counts_ref,
                 pos_ref, bexp_ref, arows_ref):
    x = x_ref[...]
    logits = jnp.dot(x, wr_ref[...], preferred_element_type=jnp.float32)
    logits = logits + br_ref[...]
    logits_ref[...] = logits

    m = jnp.max(logits, axis=1, keepdims=True)
    e = jnp.exp(logits - m)
    probs = e / jnp.sum(e, axis=1, keepdims=True)
    probs_ref[...] = probs

    lane = jax.lax.broadcasted_iota(jnp.int32, logits.shape, 1)
    is_max = logits == jnp.max(logits, axis=1, keepdims=True)
    routes = jnp.min(jnp.where(is_max, lane, NUM_EXPERTS), axis=1,
                     keepdims=True)
    routes_ref[...] = routes

    onehot = (lane == routes).astype(jnp.float32)
    counts = jnp.sum(onehot, axis=0, keepdims=True)
    counts_ref[...] = counts

    c = onehot
    s = 1
    while s < N_TOK:
        shifted = jnp.concatenate(
            [jnp.zeros((s, NUM_EXPERTS), jnp.float32), c[:-s, :]], axis=0)
        c = c + shifted
        s *= 2
    rank = jnp.sum(c * onehot, axis=1, keepdims=True) - 1.0

    nb = jnp.ceil(counts / BLK).astype(jnp.int32)
    cb = nb
    s = 1
    while s < NUM_EXPERTS:
        lane8 = jax.lax.broadcasted_iota(jnp.int32, cb.shape, 1)
        rolled = pltpu.roll(cb, s, 1)
        cb = cb + jnp.where(lane8 >= s, rolled, 0)
        s *= 2
    excl = cb - nb
    pad_off = excl * BLK

    off_tok = jnp.sum(onehot * pad_off.astype(jnp.float32), axis=1,
                      keepdims=True)
    pos_ref[...] = (off_tok + rank).astype(jnp.int32)

    bid = jax.lax.broadcasted_iota(jnp.int32, (NBLK, NUM_EXPERTS), 0)
    ge = (bid >= jnp.broadcast_to(cb, (NBLK, NUM_EXPERTS))).astype(jnp.int32)
    bexp = jnp.minimum(jnp.sum(ge, axis=1, keepdims=True),
                       NUM_EXPERTS - 1)
    bexp_ref[...] = bexp

    lane_b = jax.lax.broadcasted_iota(jnp.int32, (NBLK, NUM_EXPERTS), 1)
    bhot = (lane_b == bexp).astype(jnp.float32)
    cnt_b = jnp.sum(bhot * counts, axis=1, keepdims=True)
    excl_b = jnp.sum(bhot * excl.astype(jnp.float32), axis=1,
                     keepdims=True)
    before = (bid[:, :1] - excl_b) * BLK
    arows_ref[...] = jnp.clip(cnt_b - before, 0, BLK).astype(jnp.int32)


def _router(x, Wr, br, interpret=False):
    out_types = (
        jax.ShapeDtypeStruct((N_TOK, NUM_EXPERTS), jnp.float32),
        jax.ShapeDtypeStruct((N_TOK, NUM_EXPERTS), jnp.float32),
        jax.ShapeDtypeStruct((N_TOK, 1), jnp.int32),
        jax.ShapeDtypeStruct((1, NUM_EXPERTS), jnp.float32),
        jax.ShapeDtypeStruct((N_TOK, 1), jnp.int32),
        jax.ShapeDtypeStruct((NBLK, 1), jnp.int32),
        jax.ShapeDtypeStruct((NBLK, 1), jnp.int32),
    )
    return pl.pallas_call(_router_body, out_shape=out_types,
                          interpret=interpret)(x, Wr, br.reshape(1, -1))


SUB = 256


def _expert_body(bexp_ref, arows_ref, xs_ref, w1_ref, b1_ref, w2_ref, b2_ref,
                 ys_ref, w1b_ref, w2b_ref):
    b = pl.program_id(0)
    active = arows_ref[b]
    prev = bexp_ref[jnp.maximum(b - 1, 0)]
    switch = jnp.logical_or(b == 0, bexp_ref[b] != prev)

    @pl.when(jnp.logical_and(switch, active > 0))
    def _cast_weights():
        w1b_ref[...] = w1_ref[0].astype(jnp.bfloat16)
        w2b_ref[...] = w2_ref[0].astype(jnp.bfloat16)

    for j in range(BLK // SUB):
        @pl.when(active > j * SUB)
        def _compute(j=j):
            xb = xs_ref[pl.ds(j * SUB, SUB), :].astype(jnp.bfloat16)
            h = jnp.dot(xb, w1b_ref[...], preferred_element_type=jnp.float32)
            h = jnp.maximum(h + b1_ref[0], 0.0).astype(jnp.bfloat16)
            y = jnp.dot(h, w2b_ref[...], preferred_element_type=jnp.float32)
            ys_ref[pl.ds(j * SUB, SUB), :] = y + b2_ref[0]


def _experts(xs, W1, b1, W2, b2, bexp, arows, interpret=False):
    grid_spec = pltpu.PrefetchScalarGridSpec(
        num_scalar_prefetch=2,
        grid=(NBLK,),
        in_specs=[
            pl.BlockSpec((BLK, D_MODEL), lambda b, be, ar: (b, 0)),
            pl.BlockSpec((1, D_MODEL, D_FF), lambda b, be, ar: (be[b], 0, 0)),
            pl.BlockSpec((1, 1, D_FF), lambda b, be, ar: (be[b], 0, 0)),
            pl.BlockSpec((1, D_FF, D_OUT), lambda b, be, ar: (be[b], 0, 0)),
            pl.BlockSpec((1, 1, D_OUT), lambda b, be, ar: (be[b], 0, 0)),
        ],
        out_specs=pl.BlockSpec((BLK, D_OUT), lambda b, be, ar: (b, 0)),
        scratch_shapes=[
            pltpu.VMEM((D_MODEL, D_FF), jnp.bfloat16),
            pltpu.VMEM((D_FF, D_OUT), jnp.bfloat16),
        ],
    )
    return pl.pallas_call(
        _expert_body,
        grid_spec=grid_spec,
        out_shape=jax.ShapeDtypeStruct((NPAD, D_OUT), jnp.float32),
        interpret=interpret,
    )(bexp, arows, xs, W1, b1.reshape(NUM_EXPERTS, 1, D_FF), W2,
      b2.reshape(NUM_EXPERTS, 1, D_OUT))


NW = 32
ROWS_PER_W = N_TOK // NW


def _dispatch(x, pos):
    @functools.partial(
        pl.kernel,
        mesh=plsc.VectorSubcoreMesh(core_axis_name="c", subcore_axis_name="s"),
        out_type=jax.ShapeDtypeStruct((NPAD, D_MODEL), jnp.float32),
        scratch_types=[
            pltpu.VMEM((ROWS_PER_W,), jnp.int32),
            pltpu.VMEM((ROWS_PER_W, D_MODEL), jnp.float32),
        ],
    )
    def k(x_hbm, pos_hbm, xs_hbm, idx_v, rows_v):
        wid = jax.lax.axis_index("s") * 2 + jax.lax.axis_index("c")
        base = wid * ROWS_PER_W
        pltpu.sync_copy(pos_hbm.at[pl.ds(base, ROWS_PER_W)], idx_v)
        pltpu.sync_copy(x_hbm.at[pl.ds(base, ROWS_PER_W)], rows_v)
        pltpu.sync_copy(rows_v, xs_hbm.at[idx_v])

    return k(x, pos)


def _combine(ys, pos):
    @functools.partial(
        pl.kernel,
        mesh=plsc.VectorSubcoreMesh(core_axis_name="c", subcore_axis_name="s"),
        out_type=jax.ShapeDtypeStruct((N_TOK, D_OUT), jnp.float32),
        scratch_types=[
            pltpu.VMEM((ROWS_PER_W,), jnp.int32),
            pltpu.VMEM((ROWS_PER_W, D_OUT), jnp.float32),
        ],
    )
    def k(ys_hbm, pos_hbm, out_hbm, idx_v, rows_v):
        wid = jax.lax.axis_index("s") * 2 + jax.lax.axis_index("c")
        base = wid * ROWS_PER_W
        pltpu.sync_copy(pos_hbm.at[pl.ds(base, ROWS_PER_W)], idx_v)
        pltpu.sync_copy(ys_hbm.at[idx_v], rows_v)
        pltpu.sync_copy(rows_v, out_hbm.at[pl.ds(base, ROWS_PER_W)])

    return k(ys, pos)


@functools.partial(jax.jit, static_argnames=("interpret",))
def kernel(x, Wr, br, W1, b1, W2, b2, interpret=False):
    logits, probs, routes2, counts2, pos2, bexp2, arows2 = _router(
        x, Wr, br, interpret=interpret)
    routes = routes2.reshape(N_TOK)
    counts = counts2.reshape(NUM_EXPERTS)
    pos = pos2.reshape(N_TOK)
    bexp = bexp2.reshape(NBLK)
    arows = arows2.reshape(NBLK)

    xs = _dispatch(x, pos)
    ys = _experts(xs, W1, b1, W2, b2, bexp, arows, interpret=interpret)
    out = _combine(ys, pos)

    return (out, out, probs, counts, routes, logits)


# speedup vs baseline: 1.0437x; 1.0437x over previous
import functools

import jax
import jax.numpy as jnp
from jax.experimental import pallas as pl
from jax.experimental.pallas import tpu as pltpu
from jax.experimental.pallas import tpu_sc as plsc

NUM_EXPERTS = 8
D_MODEL = 1024
D_FF = 2048
D_OUT = 1024
N_TOK = 2048

BLK = 512
NBLK = 11
NPAD = NBLK * BLK


def _router_body(x_ref, wr_ref, br_ref,
                 logits_ref, probs_ref, routes_ref, counts_ref,
                 pos_ref, bexp_ref, arows_ref, lastb_ref):
    x = x_ref[...]
    logits = jnp.dot(x, wr_ref[...], preferred_element_type=jnp.float32)
    logits = logits + br_ref[...]
    logits_ref[...] = logits

    m = jnp.max(logits, axis=1, keepdims=True)
    e = jnp.exp(logits - m)
    probs = e / jnp.sum(e, axis=1, keepdims=True)
    probs_ref[...] = probs

    lane = jax.lax.broadcasted_iota(jnp.int32, logits.shape, 1)
    is_max = logits == jnp.max(logits, axis=1, keepdims=True)
    routes = jnp.min(jnp.where(is_max, lane, NUM_EXPERTS), axis=1,
                     keepdims=True)
    routes_ref[...] = routes

    onehot = (lane == routes).astype(jnp.float32)
    counts = jnp.sum(onehot, axis=0, keepdims=True)
    counts_ref[...] = counts

    c = onehot
    s = 1
    while s < N_TOK:
        shifted = jnp.concatenate(
            [jnp.zeros((s, NUM_EXPERTS), jnp.float32), c[:-s, :]], axis=0)
        c = c + shifted
        s *= 2
    rank = jnp.sum(c * onehot, axis=1, keepdims=True) - 1.0

    nb = jnp.ceil(counts / BLK).astype(jnp.int32)
    cb = nb
    s = 1
    while s < NUM_EXPERTS:
        lane8 = jax.lax.broadcasted_iota(jnp.int32, cb.shape, 1)
        rolled = pltpu.roll(cb, s, 1)
        cb = cb + jnp.where(lane8 >= s, rolled, 0)
        s *= 2
    excl = cb - nb
    pad_off = excl * BLK

    off_tok = jnp.sum(onehot * pad_off.astype(jnp.float32), axis=1,
                      keepdims=True)
    pos_ref[...] = (off_tok + rank).astype(jnp.int32)

    bid = jax.lax.broadcasted_iota(jnp.int32, (NBLK, NUM_EXPERTS), 0)
    ge = (bid >= jnp.broadcast_to(cb, (NBLK, NUM_EXPERTS))).astype(jnp.int32)
    lane8b = jax.lax.broadcasted_iota(jnp.int32, (1, NUM_EXPERTS), 1)
    laste = jnp.max(jnp.where(nb > 0, lane8b, 0))
    bexp = jnp.minimum(jnp.sum(ge, axis=1, keepdims=True), laste)
    bexp_ref[...] = bexp
    lastb_ref[...] = jnp.sum(nb, axis=1, keepdims=True) - 1

    lane_b = jax.lax.broadcasted_iota(jnp.int32, (NBLK, NUM_EXPERTS), 1)
    bhot = (lane_b == bexp).astype(jnp.float32)
    cnt_b = jnp.sum(bhot * counts, axis=1, keepdims=True)
    excl_b = jnp.sum(bhot * excl.astype(jnp.float32), axis=1,
                     keepdims=True)
    before = (bid[:, :1] - excl_b) * BLK
    arows_ref[...] = jnp.clip(cnt_b - before, 0, BLK).astype(jnp.int32)


def _router(x, Wr, br, interpret=False):
    out_types = (
        jax.ShapeDtypeStruct((N_TOK, NUM_EXPERTS), jnp.float32),
        jax.ShapeDtypeStruct((N_TOK, NUM_EXPERTS), jnp.float32),
        jax.ShapeDtypeStruct((N_TOK, 1), jnp.int32),
        jax.ShapeDtypeStruct((1, NUM_EXPERTS), jnp.float32),
        jax.ShapeDtypeStruct((N_TOK, 1), jnp.int32),
        jax.ShapeDtypeStruct((NBLK, 1), jnp.int32),
        jax.ShapeDtypeStruct((NBLK, 1), jnp.int32),
        jax.ShapeDtypeStruct((1, 1), jnp.int32),
    )
    return pl.pallas_call(_router_body, out_shape=out_types,
                          interpret=interpret)(x, Wr, br.reshape(1, -1))


SUB = 128


def _expert_body(bexp_ref, arows_ref, lastb_ref, xs_ref, w1_ref, b1_ref,
                 w2_ref, b2_ref, ys_ref, w1b_ref, w2b_ref):
    b = pl.program_id(0)
    active = arows_ref[b]
    prev = bexp_ref[jnp.maximum(b - 1, 0)]
    switch = jnp.logical_or(b == 0, bexp_ref[b] != prev)

    @pl.when(jnp.logical_and(switch, active > 0))
    def _cast_weights():
        w1b_ref[...] = w1_ref[0].astype(jnp.bfloat16)
        w2b_ref[...] = w2_ref[0].astype(jnp.bfloat16)

    for j in range(BLK // SUB):
        @pl.when(active > j * SUB)
        def _compute(j=j):
            xb = xs_ref[pl.ds(j * SUB, SUB), :].astype(jnp.bfloat16)
            h = jnp.dot(xb, w1b_ref[...], preferred_element_type=jnp.float32)
            h = jnp.maximum(h + b1_ref[0], 0.0).astype(jnp.bfloat16)
            y = jnp.dot(h, w2b_ref[...], preferred_element_type=jnp.float32)
            ys_ref[pl.ds(j * SUB, SUB), :] = y + b2_ref[0]


def _experts(xs, W1, b1, W2, b2, bexp, arows, lastb, interpret=False):
    clamped = lambda b, be, ar, lb: (jnp.minimum(b, lb[0]), 0)
    grid_spec = pltpu.PrefetchScalarGridSpec(
        num_scalar_prefetch=3,
        grid=(NBLK,),
        in_specs=[
            pl.BlockSpec((BLK, D_MODEL), clamped),
            pl.BlockSpec((1, D_MODEL, D_FF), lambda b, be, ar, lb: (be[b], 0, 0)),
            pl.BlockSpec((1, 1, D_FF), lambda b, be, ar, lb: (be[b], 0, 0)),
            pl.BlockSpec((1, D_FF, D_OUT), lambda b, be, ar, lb: (be[b], 0, 0)),
            pl.BlockSpec((1, 1, D_OUT), lambda b, be, ar, lb: (be[b], 0, 0)),
        ],
        out_specs=pl.BlockSpec((BLK, D_OUT), clamped),
        scratch_shapes=[
            pltpu.VMEM((D_MODEL, D_FF), jnp.bfloat16),
            pltpu.VMEM((D_FF, D_OUT), jnp.bfloat16),
        ],
    )
    return pl.pallas_call(
        _expert_body,
        grid_spec=grid_spec,
        out_shape=jax.ShapeDtypeStruct((NPAD, D_OUT), jnp.float32),
        interpret=interpret,
    )(bexp, arows, lastb, xs, W1, b1.reshape(NUM_EXPERTS, 1, D_FF), W2,
      b2.reshape(NUM_EXPERTS, 1, D_OUT))


NW = 32
ROWS_PER_W = N_TOK // NW


def _dispatch(x, pos):
    @functools.partial(
        pl.kernel,
        mesh=plsc.VectorSubcoreMesh(core_axis_name="c", subcore_axis_name="s"),
        out_type=jax.ShapeDtypeStruct((NPAD, D_MODEL), jnp.float32),
        scratch_types=[
            pltpu.VMEM((ROWS_PER_W,), jnp.int32),
            pltpu.VMEM((ROWS_PER_W, D_MODEL), jnp.float32),
        ],
    )
    def k(x_hbm, pos_hbm, xs_hbm, idx_v, rows_v):
        wid = jax.lax.axis_index("s") * 2 + jax.lax.axis_index("c")
        base = wid * ROWS_PER_W
        pltpu.sync_copy(pos_hbm.at[pl.ds(base, ROWS_PER_W)], idx_v)
        pltpu.sync_copy(x_hbm.at[pl.ds(base, ROWS_PER_W)], rows_v)
        pltpu.sync_copy(rows_v, xs_hbm.at[idx_v])

    return k(x, pos)


def _combine(ys, pos):
    @functools.partial(
        pl.kernel,
        mesh=plsc.VectorSubcoreMesh(core_axis_name="c", subcore_axis_name="s"),
        out_type=jax.ShapeDtypeStruct((N_TOK, D_OUT), jnp.float32),
        scratch_types=[
            pltpu.VMEM((ROWS_PER_W,), jnp.int32),
            pltpu.VMEM((ROWS_PER_W, D_OUT), jnp.float32),
        ],
    )
    def k(ys_hbm, pos_hbm, out_hbm, idx_v, rows_v):
        wid = jax.lax.axis_index("s") * 2 + jax.lax.axis_index("c")
        base = wid * ROWS_PER_W
        pltpu.sync_copy(pos_hbm.at[pl.ds(base, ROWS_PER_W)], idx_v)
        pltpu.sync_copy(ys_hbm.at[idx_v], rows_v)
        pltpu.sync_copy(rows_v, out_hbm.at[pl.ds(base, ROWS_PER_W)])

    return k(ys, pos)


@functools.partial(jax.jit, static_argnames=("interpret",))
def kernel(x, Wr, br, W1, b1, W2, b2, interpret=False):
    logits, probs, routes2, counts2, pos2, bexp2, arows2, lastb2 = _router(
        x, Wr, br, interpret=interpret)
    routes = routes2.reshape(N_TOK)
    counts = counts2.reshape(NUM_EXPERTS)
    pos = pos2.reshape(N_TOK)
    bexp = bexp2.reshape(NBLK)
    arows = arows2.reshape(NBLK)
    lastb = lastb2.reshape(1)

    xs = _dispatch(x, pos)
    ys = _experts(xs, W1, b1, W2, b2, bexp, arows, lastb, interpret=interpret)
    out = _combine(ys, pos)

    return (out, out, probs, counts, routes, logits)


# speedup vs baseline: 1.0617x; 1.0173x over previous
import functools

import jax
import jax.numpy as jnp
from jax.experimental import pallas as pl
from jax.experimental.pallas import tpu as pltpu
from jax.experimental.pallas import tpu_sc as plsc

NUM_EXPERTS = 8
D_MODEL = 1024
D_FF = 2048
D_OUT = 1024
N_TOK = 2048

BLK = 512
NBLK = 11
NPAD = NBLK * BLK


def _router_body(x_ref, wr_ref, br_ref,
                 logits_ref, probs_ref, routes_ref, counts_ref,
                 pos_ref, bexp_ref, arows_ref, lastb_ref):
    x = x_ref[...]
    logits = jnp.dot(x, wr_ref[...], preferred_element_type=jnp.float32)
    logits = logits + br_ref[...]
    logits_ref[...] = logits

    m = jnp.max(logits, axis=1, keepdims=True)
    e = jnp.exp(logits - m)
    probs = e / jnp.sum(e, axis=1, keepdims=True)
    probs_ref[...] = probs

    lane = jax.lax.broadcasted_iota(jnp.int32, logits.shape, 1)
    is_max = logits == jnp.max(logits, axis=1, keepdims=True)
    routes = jnp.min(jnp.where(is_max, lane, NUM_EXPERTS), axis=1,
                     keepdims=True)
    routes_ref[...] = routes

    onehot = (lane == routes).astype(jnp.float32)
    counts = jnp.sum(onehot, axis=0, keepdims=True)
    counts_ref[...] = counts

    c = onehot
    s = 1
    while s < N_TOK:
        shifted = jnp.concatenate(
            [jnp.zeros((s, NUM_EXPERTS), jnp.float32), c[:-s, :]], axis=0)
        c = c + shifted
        s *= 2
    rank = jnp.sum(c * onehot, axis=1, keepdims=True) - 1.0

    nb = jnp.ceil(counts / BLK).astype(jnp.int32)
    cb = nb
    s = 1
    while s < NUM_EXPERTS:
        lane8 = jax.lax.broadcasted_iota(jnp.int32, cb.shape, 1)
        rolled = pltpu.roll(cb, s, 1)
        cb = cb + jnp.where(lane8 >= s, rolled, 0)
        s *= 2
    excl = cb - nb
    pad_off = excl * BLK

    off_tok = jnp.sum(onehot * pad_off.astype(jnp.float32), axis=1,
                      keepdims=True)
    pos_ref[...] = (off_tok + rank).astype(jnp.int32)

    bid = jax.lax.broadcasted_iota(jnp.int32, (NBLK, NUM_EXPERTS), 0)
    ge = (bid >= jnp.broadcast_to(cb, (NBLK, NUM_EXPERTS))).astype(jnp.int32)
    lane8b = jax.lax.broadcasted_iota(jnp.int32, (1, NUM_EXPERTS), 1)
    laste = jnp.max(jnp.where(nb > 0, lane8b, 0))
    bexp = jnp.minimum(jnp.sum(ge, axis=1, keepdims=True), laste)
    bexp_ref[...] = bexp
    lastb_ref[...] = jnp.sum(nb, axis=1, keepdims=True) - 1

    lane_b = jax.lax.broadcasted_iota(jnp.int32, (NBLK, NUM_EXPERTS), 1)
    bhot = (lane_b == bexp).astype(jnp.float32)
    cnt_b = jnp.sum(bhot * counts, axis=1, keepdims=True)
    excl_b = jnp.sum(bhot * excl.astype(jnp.float32), axis=1,
                     keepdims=True)
    before = (bid[:, :1] - excl_b) * BLK
    arows_ref[...] = jnp.clip(cnt_b - before, 0, BLK).astype(jnp.int32)


def _router(x, Wr, br, interpret=False):
    out_types = (
        jax.ShapeDtypeStruct((N_TOK, NUM_EXPERTS), jnp.float32),
        jax.ShapeDtypeStruct((N_TOK, NUM_EXPERTS), jnp.float32),
        jax.ShapeDtypeStruct((N_TOK, 1), jnp.int32),
        jax.ShapeDtypeStruct((1, NUM_EXPERTS), jnp.float32),
        jax.ShapeDtypeStruct((N_TOK, 1), jnp.int32),
        jax.ShapeDtypeStruct((NBLK, 1), jnp.int32),
        jax.ShapeDtypeStruct((NBLK, 1), jnp.int32),
        jax.ShapeDtypeStruct((1, 1), jnp.int32),
    )
    return pl.pallas_call(_router_body, out_shape=out_types,
                          interpret=interpret)(x, Wr, br.reshape(1, -1))


SUB = 128


def _expert_body(bexp_ref, arows_ref, lastb_ref, xs_ref, w1_ref, b1_ref,
                 w2_ref, b2_ref, ys_ref, w1b_ref, w2b_ref):
    b = pl.program_id(0)
    active = arows_ref[b]
    prev = bexp_ref[jnp.maximum(b - 1, 0)]
    switch = jnp.logical_or(b == 0, bexp_ref[b] != prev)

    @pl.when(jnp.logical_and(switch, active > 0))
    def _cast_weights():
        w1b_ref[...] = w1_ref[0].astype(jnp.bfloat16)
        w2b_ref[...] = w2_ref[0].astype(jnp.bfloat16)

    def mlp(off, rows):
        xb = xs_ref[pl.ds(off, rows), :].astype(jnp.bfloat16)
        h = jnp.dot(xb, w1b_ref[...], preferred_element_type=jnp.float32)
        h = jnp.maximum(h + b1_ref[0], 0.0).astype(jnp.bfloat16)
        y = jnp.dot(h, w2b_ref[...], preferred_element_type=jnp.float32)
        ys_ref[pl.ds(off, rows), :] = y + b2_ref[0]

    nsub = (active + SUB - 1) // SUB
    pl.when(nsub == 1)(lambda: mlp(0, SUB))
    pl.when(nsub == 2)(lambda: mlp(0, 2 * SUB))
    pl.when(nsub == 3)(lambda: mlp(0, 2 * SUB))
    pl.when(nsub == 3)(lambda: mlp(2 * SUB, SUB))
    pl.when(nsub == 4)(lambda: mlp(0, 4 * SUB))


def _experts(xs, W1, b1, W2, b2, bexp, arows, lastb, interpret=False):
    clamped = lambda b, be, ar, lb: (jnp.minimum(b, lb[0]), 0)
    grid_spec = pltpu.PrefetchScalarGridSpec(
        num_scalar_prefetch=3,
        grid=(NBLK,),
        in_specs=[
            pl.BlockSpec((BLK, D_MODEL), clamped),
            pl.BlockSpec((1, D_MODEL, D_FF), lambda b, be, ar, lb: (be[b], 0, 0)),
            pl.BlockSpec((1, 1, D_FF), lambda b, be, ar, lb: (be[b], 0, 0)),
            pl.BlockSpec((1, D_FF, D_OUT), lambda b, be, ar, lb: (be[b], 0, 0)),
            pl.BlockSpec((1, 1, D_OUT), lambda b, be, ar, lb: (be[b], 0, 0)),
        ],
        out_specs=pl.BlockSpec((BLK, D_OUT), clamped),
        scratch_shapes=[
            pltpu.VMEM((D_MODEL, D_FF), jnp.bfloat16),
            pltpu.VMEM((D_FF, D_OUT), jnp.bfloat16),
        ],
    )
    return pl.pallas_call(
        _expert_body,
        grid_spec=grid_spec,
        out_shape=jax.ShapeDtypeStruct((NPAD, D_OUT), jnp.float32),
        interpret=interpret,
    )(bexp, arows, lastb, xs, W1, b1.reshape(NUM_EXPERTS, 1, D_FF), W2,
      b2.reshape(NUM_EXPERTS, 1, D_OUT))


NW = 32
ROWS_PER_W = N_TOK // NW


def _dispatch(x, pos):
    @functools.partial(
        pl.kernel,
        mesh=plsc.VectorSubcoreMesh(core_axis_name="c", subcore_axis_name="s"),
        out_type=jax.ShapeDtypeStruct((NPAD, D_MODEL), jnp.float32),
        scratch_types=[
            pltpu.VMEM((ROWS_PER_W,), jnp.int32),
            pltpu.VMEM((ROWS_PER_W, D_MODEL), jnp.float32),
        ],
    )
    def k(x_hbm, pos_hbm, xs_hbm, idx_v, rows_v):
        wid = jax.lax.axis_index("s") * 2 + jax.lax.axis_index("c")
        base = wid * ROWS_PER_W
        pltpu.sync_copy(pos_hbm.at[pl.ds(base, ROWS_PER_W)], idx_v)
        pltpu.sync_copy(x_hbm.at[pl.ds(base, ROWS_PER_W)], rows_v)
        pltpu.sync_copy(rows_v, xs_hbm.at[idx_v])

    return k(x, pos)


def _combine(ys, pos):
    @functools.partial(
        pl.kernel,
        mesh=plsc.VectorSubcoreMesh(core_axis_name="c", subcore_axis_name="s"),
        out_type=jax.ShapeDtypeStruct((N_TOK, D_OUT), jnp.float32),
        scratch_types=[
            pltpu.VMEM((ROWS_PER_W,), jnp.int32),
            pltpu.VMEM((ROWS_PER_W, D_OUT), jnp.float32),
        ],
    )
    def k(ys_hbm, pos_hbm, out_hbm, idx_v, rows_v):
        wid = jax.lax.axis_index("s") * 2 + jax.lax.axis_index("c")
        base = wid * ROWS_PER_W
        pltpu.sync_copy(pos_hbm.at[pl.ds(base, ROWS_PER_W)], idx_v)
        pltpu.sync_copy(ys_hbm.at[idx_v], rows_v)
        pltpu.sync_copy(rows_v, out_hbm.at[pl.ds(base, ROWS_PER_W)])

    return k(ys, pos)


@functools.partial(jax.jit, static_argnames=("interpret",))
def kernel(x, Wr, br, W1, b1, W2, b2, interpret=False):
    logits, probs, routes2, counts2, pos2, bexp2, arows2, lastb2 = _router(
        x, Wr, br, interpret=interpret)
    routes = routes2.reshape(N_TOK)
    counts = counts2.reshape(NUM_EXPERTS)
    pos = pos2.reshape(N_TOK)
    bexp = bexp2.reshape(NBLK)
    arows = arows2.reshape(NBLK)
    lastb = lastb2.reshape(1)

    xs = _dispatch(x, pos)
    ys = _experts(xs, W1, b1, W2, b2, bexp, arows, lastb, interpret=interpret)
    out = _combine(ys, pos)

    return (out, out, probs, counts, routes, logits)


# speedup vs baseline: 1.0627x; 1.0009x over previous
import functools

import jax
import jax.numpy as jnp
from jax.experimental import pallas as pl
from jax.experimental.pallas import tpu as pltpu
from jax.experimental.pallas import tpu_sc as plsc

NUM_EXPERTS = 8
D_MODEL = 1024
D_FF = 2048
D_OUT = 1024
N_TOK = 2048

BLK = 512
NBLK = 11
NPAD = NBLK * BLK


def _router_body(x_ref, wr_ref, br_ref,
                 logits_ref, probs_ref, routes_ref, counts_ref,
                 pos_ref, bexp_ref, arows_ref, lastb_ref):
    x = x_ref[...]
    logits = jnp.dot(x, wr_ref[...], preferred_element_type=jnp.float32)
    logits = logits + br_ref[...]
    logits_ref[...] = logits

    m = jnp.max(logits, axis=1, keepdims=True)
    e = jnp.exp(logits - m)
    probs = e / jnp.sum(e, axis=1, keepdims=True)
    probs_ref[...] = probs

    lane = jax.lax.broadcasted_iota(jnp.int32, logits.shape, 1)
    is_max = logits == jnp.max(logits, axis=1, keepdims=True)
    routes = jnp.min(jnp.where(is_max, lane, NUM_EXPERTS), axis=1,
                     keepdims=True)
    routes_ref[...] = routes

    onehot = (lane == routes).astype(jnp.float32)
    counts = jnp.sum(onehot, axis=0, keepdims=True)
    counts_ref[...] = counts

    c = onehot
    s = 1
    while s < N_TOK:
        shifted = jnp.concatenate(
            [jnp.zeros((s, NUM_EXPERTS), jnp.float32), c[:-s, :]], axis=0)
        c = c + shifted
        s *= 2
    rank = jnp.sum(c * onehot, axis=1, keepdims=True) - 1.0

    nb = jnp.ceil(counts / BLK).astype(jnp.int32)
    cb = nb
    s = 1
    while s < NUM_EXPERTS:
        lane8 = jax.lax.broadcasted_iota(jnp.int32, cb.shape, 1)
        rolled = pltpu.roll(cb, s, 1)
        cb = cb + jnp.where(lane8 >= s, rolled, 0)
        s *= 2
    excl = cb - nb
    pad_off = excl * BLK

    off_tok = jnp.sum(onehot * pad_off.astype(jnp.float32), axis=1,
                      keepdims=True)
    pos_ref[...] = (off_tok + rank).astype(jnp.int32)

    bid = jax.lax.broadcasted_iota(jnp.int32, (NBLK, NUM_EXPERTS), 0)
    ge = (bid >= jnp.broadcast_to(cb, (NBLK, NUM_EXPERTS))).astype(jnp.int32)
    lane8b = jax.lax.broadcasted_iota(jnp.int32, (1, NUM_EXPERTS), 1)
    laste = jnp.max(jnp.where(nb > 0, lane8b, 0))
    bexp = jnp.minimum(jnp.sum(ge, axis=1, keepdims=True), laste)
    bexp_ref[...] = bexp
    lastb_ref[...] = jnp.sum(nb, axis=1, keepdims=True) - 1

    lane_b = jax.lax.broadcasted_iota(jnp.int32, (NBLK, NUM_EXPERTS), 1)
    bhot = (lane_b == bexp).astype(jnp.float32)
    cnt_b = jnp.sum(bhot * counts, axis=1, keepdims=True)
    excl_b = jnp.sum(bhot * excl.astype(jnp.float32), axis=1,
                     keepdims=True)
    before = (bid[:, :1] - excl_b) * BLK
    arows_ref[...] = jnp.clip(cnt_b - before, 0, BLK).astype(jnp.int32)


def _router(x, Wr, br):
    out_types = (
        jax.ShapeDtypeStruct((N_TOK, NUM_EXPERTS), jnp.float32),
        jax.ShapeDtypeStruct((N_TOK, NUM_EXPERTS), jnp.float32),
        jax.ShapeDtypeStruct((N_TOK, 1), jnp.int32),
        jax.ShapeDtypeStruct((1, NUM_EXPERTS), jnp.float32),
        jax.ShapeDtypeStruct((N_TOK, 1), jnp.int32),
        jax.ShapeDtypeStruct((NBLK, 1), jnp.int32),
        jax.ShapeDtypeStruct((NBLK, 1), jnp.int32),
        jax.ShapeDtypeStruct((1, 1), jnp.int32),
    )
    return pl.pallas_call(_router_body,
                          out_shape=out_types)(x, Wr, br.reshape(1, -1))


SUB = 128


def _expert_body(bexp_ref, arows_ref, lastb_ref, xs_ref, w1_ref, b1_ref,
                 w2_ref, b2_ref, ys_ref, w1b_ref, w2b_ref):
    b = pl.program_id(0)
    active = arows_ref[b]
    prev = bexp_ref[jnp.maximum(b - 1, 0)]
    switch = jnp.logical_or(b == 0, bexp_ref[b] != prev)

    @pl.when(jnp.logical_and(switch, active > 0))
    def _cast_weights():
        w1b_ref[...] = w1_ref[0].astype(jnp.bfloat16)
        w2b_ref[...] = w2_ref[0].astype(jnp.bfloat16)

    def mlp(off, rows):
        xb = xs_ref[pl.ds(off, rows), :].astype(jnp.bfloat16)
        h = jnp.dot(xb, w1b_ref[...], preferred_element_type=jnp.float32)
        h = jnp.maximum(h + b1_ref[0], 0.0).astype(jnp.bfloat16)
        y = jnp.dot(h, w2b_ref[...], preferred_element_type=jnp.float32)
        ys_ref[pl.ds(off, rows), :] = y + b2_ref[0]

    nsub = (active + SUB - 1) // SUB
    pl.when(nsub == 1)(lambda: mlp(0, SUB))
    pl.when(nsub == 2)(lambda: mlp(0, 2 * SUB))
    pl.when(nsub == 3)(lambda: mlp(0, 2 * SUB))
    pl.when(nsub == 3)(lambda: mlp(2 * SUB, SUB))
    pl.when(nsub == 4)(lambda: mlp(0, 4 * SUB))


def _experts(xs, W1, b1, W2, b2, bexp, arows, lastb):
    clamped = lambda b, be, ar, lb: (jnp.minimum(b, lb[0]), 0)
    grid_spec = pltpu.PrefetchScalarGridSpec(
        num_scalar_prefetch=3,
        grid=(NBLK,),
        in_specs=[
            pl.BlockSpec((BLK, D_MODEL), clamped),
            pl.BlockSpec((1, D_MODEL, D_FF), lambda b, be, ar, lb: (be[b], 0, 0)),
            pl.BlockSpec((1, 1, D_FF), lambda b, be, ar, lb: (be[b], 0, 0)),
            pl.BlockSpec((1, D_FF, D_OUT), lambda b, be, ar, lb: (be[b], 0, 0)),
            pl.BlockSpec((1, 1, D_OUT), lambda b, be, ar, lb: (be[b], 0, 0)),
        ],
        out_specs=pl.BlockSpec((BLK, D_OUT), clamped),
        scratch_shapes=[
            pltpu.VMEM((D_MODEL, D_FF), jnp.bfloat16),
            pltpu.VMEM((D_FF, D_OUT), jnp.bfloat16),
        ],
    )
    return pl.pallas_call(
        _expert_body,
        grid_spec=grid_spec,
        out_shape=jax.ShapeDtypeStruct((NPAD, D_OUT), jnp.float32),
    )(bexp, arows, lastb, xs, W1, b1.reshape(NUM_EXPERTS, 1, D_FF), W2,
      b2.reshape(NUM_EXPERTS, 1, D_OUT))


NW = 32
ROWS_PER_W = N_TOK // NW


def _dispatch(x, pos):
    @functools.partial(
        pl.kernel,
        mesh=plsc.VectorSubcoreMesh(core_axis_name="c", subcore_axis_name="s"),
        out_type=jax.ShapeDtypeStruct((NPAD, D_MODEL), jnp.float32),
        scratch_types=[
            pltpu.VMEM((ROWS_PER_W,), jnp.int32),
            pltpu.VMEM((ROWS_PER_W, D_MODEL), jnp.float32),
        ],
    )
    def k(x_hbm, pos_hbm, xs_hbm, idx_v, rows_v):
        wid = jax.lax.axis_index("s") * 2 + jax.lax.axis_index("c")
        base = wid * ROWS_PER_W
        pltpu.sync_copy(pos_hbm.at[pl.ds(base, ROWS_PER_W)], idx_v)
        pltpu.sync_copy(x_hbm.at[pl.ds(base, ROWS_PER_W)], rows_v)
        pltpu.sync_copy(rows_v, xs_hbm.at[idx_v])

    return k(x, pos)


def _combine(ys, pos):
    @functools.partial(
        pl.kernel,
        mesh=plsc.VectorSubcoreMesh(core_axis_name="c", subcore_axis_name="s"),
        out_type=jax.ShapeDtypeStruct((N_TOK, D_OUT), jnp.float32),
        scratch_types=[
            pltpu.VMEM((ROWS_PER_W,), jnp.int32),
            pltpu.VMEM((ROWS_PER_W, D_OUT), jnp.float32),
        ],
    )
    def k(ys_hbm, pos_hbm, out_hbm, idx_v, rows_v):
        wid = jax.lax.axis_index("s") * 2 + jax.lax.axis_index("c")
        base = wid * ROWS_PER_W
        pltpu.sync_copy(pos_hbm.at[pl.ds(base, ROWS_PER_W)], idx_v)
        pltpu.sync_copy(ys_hbm.at[idx_v], rows_v)
        pltpu.sync_copy(rows_v, out_hbm.at[pl.ds(base, ROWS_PER_W)])

    return k(ys, pos)


@jax.jit
def kernel(x, Wr, br, W1, b1, W2, b2):
    logits, probs, routes2, counts2, pos2, bexp2, arows2, lastb2 = _router(x, Wr, br)
    routes = routes2.reshape(N_TOK)
    counts = counts2.reshape(NUM_EXPERTS)
    pos = pos2.reshape(N_TOK)
    bexp = bexp2.reshape(NBLK)
    arows = arows2.reshape(NBLK)
    lastb = lastb2.reshape(1)

    xs = _dispatch(x, pos)
    ys = _experts(xs, W1, b1, W2, b2, bexp, arows, lastb)
    out = _combine(ys, pos)

    return (out, out, probs, counts, routes, logits)
